# baseline (speedup 1.0000x reference)
import jax
import jax.numpy as jnp
from jax.experimental import pallas as pl
from jax.experimental.pallas import tpu as pltpu

_NB2 = 2
_C = 2


def _moe_kernel(idx_ref, x_ref, w_hbm_a, w_hbm_b, b_ref, out_ref,
                w_buf_a, w_buf_b, sem_a, sem_b):
    E, D, O = w_hbm_a.shape
    DC = D // _C
    TOT = E * _C
    streams = ((w_hbm_a, w_buf_a, sem_a), (w_hbm_b, w_buf_b, sem_b))

    def make_copy(t):
        w_hbm, buf, sem = streams[t % 2]
        slot = (t // 2) % _NB2
        e = t // _C
        c = t % _C
        return pltpu.make_async_copy(
            w_hbm.at[e, pl.ds(c * DC, DC), :],
            buf.at[slot],
            sem.at[slot],
        )

    for t in range(2 * _NB2):
        make_copy(t).start()

    for e in range(E):
        mask = (idx_ref[...] == e).astype(jnp.float32)
        part = mask * b_ref[e]
        for c in range(_C):
            t = e * _C + c
            buf = streams[t % 2][1]
            slot = (t // 2) % _NB2
            make_copy(t).wait()
            xm = x_ref[:, c * DC:(c + 1) * DC] * mask
            part = part + jnp.dot(
                xm, buf[slot], preferred_element_type=jnp.float32
            )
            if t + 2 * _NB2 < TOT:
                make_copy(t + 2 * _NB2).start()
        if e == 0:
            out_ref[...] = part
        else:
            out_ref[...] += part


def kernel(inputs, indices, w, b):
    N, D = inputs.shape
    E, _, O = w.shape
    idx2d = indices.astype(jnp.int32).reshape(N, 1)
    b3d = b.reshape(E, 1, O)

    return pl.pallas_call(
        _moe_kernel,
        in_specs=[
            pl.BlockSpec(memory_space=pltpu.VMEM),
            pl.BlockSpec(memory_space=pltpu.VMEM),
            pl.BlockSpec(memory_space=pl.ANY),
            pl.BlockSpec(memory_space=pl.ANY),
            pl.BlockSpec(memory_space=pltpu.VMEM),
        ],
        out_specs=pl.BlockSpec(memory_space=pltpu.VMEM),
        out_shape=jax.ShapeDtypeStruct((N, O), jnp.float32),
        scratch_shapes=[
            pltpu.VMEM((_NB2, D // _C, O), jnp.float32),
            pltpu.VMEM((_NB2, D // _C, O), jnp.float32),
            pltpu.SemaphoreType.DMA((_NB2,)),
            pltpu.SemaphoreType.DMA((_NB2,)),
        ],
    )(idx2d, inputs, w, w, b3d)


# speedup vs baseline: 1.0054x; 1.0054x over previous
import jax
import jax.numpy as jnp
from jax.experimental import pallas as pl
from jax.experimental.pallas import tpu as pltpu

_NBUF = 4
_C = 2


def _moe_kernel(idx_ref, x_ref, w_hbm, b_ref, out_ref, w_buf, sem):
    E, D, O = w_hbm.shape
    DC = D // _C
    TOT = E * _C

    def make_copy(t, slot):
        e = t // _C
        c = jax.lax.rem(t, _C)
        return pltpu.make_async_copy(
            w_hbm.at[e, pl.ds(c * DC, DC), :],
            w_buf.at[slot],
            sem.at[slot],
        )

    for s in range(_NBUF):
        make_copy(s, s).start()

    for e in range(E):
        mask = (idx_ref[...] == e).astype(jnp.float32)
        part = mask * b_ref[e]
        for c in range(_C):
            t = e * _C + c
            s = t % _NBUF
            make_copy(t, s).wait()
            xm = x_ref[:, c * DC:(c + 1) * DC] * mask
            part = part + jnp.dot(
                xm, w_buf[s], preferred_element_type=jnp.float32
            )
            if t + _NBUF < TOT:
                make_copy(t + _NBUF, s).start()
        if e == 0:
            out_ref[...] = part
        else:
            out_ref[...] += part


def kernel(inputs, indices, w, b):
    N, D = inputs.shape
    E, _, O = w.shape
    idx2d = indices.astype(jnp.int32).reshape(N, 1)
    b3d = b.reshape(E, 1, O)

    return pl.pallas_call(
        _moe_kernel,
        in_specs=[
            pl.BlockSpec(memory_space=pltpu.VMEM),
            pl.BlockSpec(memory_space=pltpu.VMEM),
            pl.BlockSpec(memory_space=pl.ANY),
            pl.BlockSpec(memory_space=pltpu.VMEM),
        ],
        out_specs=pl.BlockSpec(memory_space=pltpu.VMEM),
        out_shape=jax.ShapeDtypeStruct((N, O), jnp.float32),
        scratch_shapes=[
            pltpu.VMEM((_NBUF, D // _C, O), jnp.float32),
            pltpu.SemaphoreType.DMA((_NBUF,)),
        ],
    )(idx2d, inputs, w, b3d)
